# baseline (speedup 1.0000x reference)
import functools

import jax
import jax.numpy as jnp
from jax import lax
from jax.experimental import pallas as pl
from jax.experimental.pallas import tpu as pltpu
from jax.experimental.pallas import tpu_sc as plsc

S = 8
H = 256
E = S * S
NUM_REL = 2 * S * S
TC_E = 32
SC_E = E - TC_E
L = 16


def _sc_body(et_ref, w_ref, gf_ref, out_ref, idx_v, w_v, x_v, msg_v, sem):
    wid = lax.axis_index("s") * 2 + lax.axis_index("c")
    e = TC_E + wid
    src = e // S
    pltpu.sync_copy(et_ref.at[wid], idx_v)
    pltpu.async_copy(w_ref.at[idx_v.at[pl.ds(0, 1)]], w_v, sem).wait()
    pltpu.sync_copy(gf_ref.at[src], x_v)

    def hstep(h, accs):
        xb = plsc.load_gather(x_v, [jnp.full((L,), h, jnp.int32)])
        return tuple(accs[kc] + xb * w_v[0, pl.ds(h * H + kc * L, L)]
                     for kc in range(H // L))

    accs = lax.fori_loop(
        0, H, hstep,
        tuple(jnp.zeros((L,), jnp.float32) for _ in range(H // L)))
    for kc in range(H // L):
        msg_v[pl.ds(kc * L, L)] = accs[kc]
    pltpu.sync_copy(msg_v, out_ref.at[wid])


def _sc_msgs(et_pad, w2d, gf):
    mesh = plsc.VectorSubcoreMesh(core_axis_name="c", subcore_axis_name="s",
                                  num_cores=2, num_subcores=16)
    return pl.kernel(
        _sc_body,
        out_type=jax.ShapeDtypeStruct((SC_E, H), jnp.float32),
        mesh=mesh,
        scratch_types=[
            pltpu.VMEM((S,), jnp.int32),
            pltpu.VMEM((1, H * H), jnp.float32),
            pltpu.VMEM((H,), jnp.float32),
            pltpu.VMEM((H,), jnp.float32),
            pltpu.SemaphoreType.DMA,
        ],
        compiler_params=pltpu.CompilerParams(needs_layout_passes=False),
    )(et_pad, w2d, gf)


def _tc_body(et_ref, gf_ref, wq_ref, wk_ref, *rest):
    w_refs = rest[:TC_E]
    root_ref, rb_ref, part_ref, attn_ref = rest[TC_E:]

    gf = gf_ref[...]
    q = jnp.dot(gf, wq_ref[...], preferred_element_type=jnp.float32)
    k = jnp.dot(gf, wk_ref[...], preferred_element_type=jnp.float32)
    scores = jnp.dot(q, k.T, preferred_element_type=jnp.float32) * (1.0 / 16.0)
    scores = scores - jnp.max(scores, axis=-1, keepdims=True)
    ex = jnp.exp(scores)
    attn = ex / jnp.sum(ex, axis=-1, keepdims=True)
    attn_ref[...] = attn
    wgf = (attn[:, :, None] * gf[:, None, :]).reshape(E, H)

    msgs = [jnp.dot(wgf[e:e + 1, :], w_refs[e][0],
                    preferred_element_type=jnp.float32) for e in range(TC_E)]
    x1 = (jnp.dot(gf, root_ref[...], preferred_element_type=jnp.float32)
          + rb_ref[...])
    for b in range(TC_E // S):
        x1 = x1 + jnp.concatenate(msgs[S * b:S * (b + 1)], axis=0)
    part_ref[...] = x1


def _combine_body(part_ref, attn_ref, raw_ref, gf_ref, lrel_ref, lroot_ref,
                  gb_ref, out_ref):
    gf = gf_ref[...]
    attn_t = attn_ref[...].T
    x1 = part_ref[...]
    for b in range(SC_E // S):
        x1 = x1 + attn_t[:, 4 + b:5 + b] * raw_ref[S * b:S * (b + 1), :]
    agg = jnp.broadcast_to(jnp.sum(x1, axis=0, keepdims=True), (S, H))
    out2 = (jnp.dot(agg, lrel_ref[...], preferred_element_type=jnp.float32)
            + jnp.dot(x1, lroot_ref[...], preferred_element_type=jnp.float32)
            + gb_ref[...])
    out_ref[:, :H] = out2
    out_ref[:, H:] = gf


def kernel(global_features, speaker, Wq, Wk, Wv, rgcn_weight, rgcn_root,
           rgcn_bias, gcn_lin_rel, gcn_lin_root, gcn_bias):
    del Wv
    spk = speaker.astype(jnp.int32)
    src = jnp.repeat(jnp.arange(S, dtype=jnp.int32), S)
    dst = jnp.tile(jnp.arange(S, dtype=jnp.int32), S)
    et = (spk[src] * S + spk[dst]) * 2 + (src >= dst).astype(jnp.int32)

    et_pad = jnp.zeros((SC_E, S), jnp.int32).at[:, 0].set(et[TC_E:])
    w2d = rgcn_weight.reshape(NUM_REL, H * H)
    raw = _sc_msgs(et_pad, w2d, global_features)

    full = lambda shape: pl.BlockSpec(shape, lambda i, et_ref: (0,) * len(shape))
    w_specs = [pl.BlockSpec((1, H, H),
                            lambda i, et_ref, e=e: (et_ref[e], 0, 0))
               for e in range(TC_E)]
    grid_spec = pltpu.PrefetchScalarGridSpec(
        num_scalar_prefetch=1,
        grid=(1,),
        in_specs=[
            full((S, H)),
            full((H, H)),
            full((H, H)),
            *w_specs,
            full((H, H)),
            full((1, H)),
        ],
        out_specs=(pl.BlockSpec((S, H), lambda i, et_ref: (0, 0)),
                   pl.BlockSpec((S, S), lambda i, et_ref: (0, 0))),
    )
    part, attn = pl.pallas_call(
        _tc_body,
        grid_spec=grid_spec,
        out_shape=(jax.ShapeDtypeStruct((S, H), jnp.float32),
                   jax.ShapeDtypeStruct((S, S), jnp.float32)),
    )(et, global_features, Wq, Wk, *([rgcn_weight] * TC_E),
      rgcn_root, rgcn_bias.reshape(1, H))

    return pl.pallas_call(
        _combine_body,
        out_shape=jax.ShapeDtypeStruct((S, 2 * H), jnp.float32),
    )(part, attn, raw, global_features, gcn_lin_rel, gcn_lin_root,
      gcn_bias.reshape(1, H))


# speedup vs baseline: 1.5605x; 1.5605x over previous
import functools

import jax
import jax.numpy as jnp
from jax import lax
from jax.experimental import pallas as pl
from jax.experimental.pallas import tpu as pltpu
from jax.experimental.pallas import tpu_sc as plsc

S = 8
H = 256
E = S * S
NUM_REL = 2 * S * S
TC_E = 32
SC_E = E - TC_E
L = 16


def _sc_body(et_ref, w_ref, gf_ref, out_ref, idx_v, w_v, x_v, msg_v, sem):
    wid = lax.axis_index("s") * 2 + lax.axis_index("c")
    e = TC_E + wid
    src = e // S
    pltpu.sync_copy(et_ref.at[wid], idx_v)
    pltpu.async_copy(w_ref.at[idx_v.at[pl.ds(0, 1)]], w_v, sem).wait()
    pltpu.sync_copy(gf_ref.at[src], x_v)

    def hstep(h, accs):
        xb = plsc.load_gather(x_v, [jnp.full((L,), h, jnp.int32)])
        return tuple(accs[kc] + xb * w_v[0, h, pl.ds(kc * L, L)]
                     for kc in range(H // L))

    accs = lax.fori_loop(
        0, H, hstep,
        tuple(jnp.zeros((L,), jnp.float32) for _ in range(H // L)))
    for kc in range(H // L):
        msg_v[pl.ds(kc * L, L)] = accs[kc]
    pltpu.sync_copy(msg_v, out_ref.at[wid])


def _sc_msgs(et_pad, w2d, gf):
    mesh = plsc.VectorSubcoreMesh(core_axis_name="c", subcore_axis_name="s",
                                  num_cores=2, num_subcores=16)
    return pl.kernel(
        _sc_body,
        out_type=jax.ShapeDtypeStruct((SC_E, H), jnp.float32),
        mesh=mesh,
        scratch_types=[
            pltpu.VMEM((S,), jnp.int32),
            pltpu.VMEM((1, H, H), jnp.float32),
            pltpu.VMEM((H,), jnp.float32),
            pltpu.VMEM((H,), jnp.float32),
            pltpu.SemaphoreType.DMA,
        ],
        compiler_params=pltpu.CompilerParams(needs_layout_passes=False),
    )(et_pad, w2d, gf)


def _tc_body(et_ref, gf_ref, wq_ref, wk_ref, *rest):
    w_refs = rest[:TC_E]
    root_ref, rb_ref, part_ref, attn_ref = rest[TC_E:]

    gf = gf_ref[...]
    q = jnp.dot(gf, wq_ref[...], preferred_element_type=jnp.float32)
    k = jnp.dot(gf, wk_ref[...], preferred_element_type=jnp.float32)
    scores = jnp.dot(q, k.T, preferred_element_type=jnp.float32) * (1.0 / 16.0)
    scores = scores - jnp.max(scores, axis=-1, keepdims=True)
    ex = jnp.exp(scores)
    attn = ex / jnp.sum(ex, axis=-1, keepdims=True)
    attn_ref[...] = attn
    wgf = (attn[:, :, None] * gf[:, None, :]).reshape(E, H)

    msgs = [jnp.dot(wgf[e:e + 1, :], w_refs[e][0],
                    preferred_element_type=jnp.float32) for e in range(TC_E)]
    x1 = (jnp.dot(gf, root_ref[...], preferred_element_type=jnp.float32)
          + rb_ref[...])
    for b in range(TC_E // S):
        x1 = x1 + jnp.concatenate(msgs[S * b:S * (b + 1)], axis=0)
    part_ref[...] = x1


def _combine_body(part_ref, attn_ref, raw_ref, gf_ref, lrel_ref, lroot_ref,
                  gb_ref, out_ref):
    gf = gf_ref[...]
    attn_t = attn_ref[...].T
    x1 = part_ref[...]
    for b in range(SC_E // S):
        x1 = x1 + attn_t[:, 4 + b:5 + b] * raw_ref[S * b:S * (b + 1), :]
    agg = jnp.broadcast_to(jnp.sum(x1, axis=0, keepdims=True), (S, H))
    out2 = (jnp.dot(agg, lrel_ref[...], preferred_element_type=jnp.float32)
            + jnp.dot(x1, lroot_ref[...], preferred_element_type=jnp.float32)
            + gb_ref[...])
    out_ref[:, :H] = out2
    out_ref[:, H:] = gf


def kernel(global_features, speaker, Wq, Wk, Wv, rgcn_weight, rgcn_root,
           rgcn_bias, gcn_lin_rel, gcn_lin_root, gcn_bias):
    del Wv
    spk = speaker.astype(jnp.int32)
    src = jnp.repeat(jnp.arange(S, dtype=jnp.int32), S)
    dst = jnp.tile(jnp.arange(S, dtype=jnp.int32), S)
    et = (spk[src] * S + spk[dst]) * 2 + (src >= dst).astype(jnp.int32)

    et_pad = jnp.zeros((SC_E, S), jnp.int32).at[:, 0].set(et[TC_E:])
    raw = _sc_msgs(et_pad, rgcn_weight, global_features)

    full = lambda shape: pl.BlockSpec(shape, lambda i, et_ref: (0,) * len(shape))
    w_specs = [pl.BlockSpec((1, H, H),
                            lambda i, et_ref, e=e: (et_ref[e], 0, 0))
               for e in range(TC_E)]
    grid_spec = pltpu.PrefetchScalarGridSpec(
        num_scalar_prefetch=1,
        grid=(1,),
        in_specs=[
            full((S, H)),
            full((H, H)),
            full((H, H)),
            *w_specs,
            full((H, H)),
            full((1, H)),
        ],
        out_specs=(pl.BlockSpec((S, H), lambda i, et_ref: (0, 0)),
                   pl.BlockSpec((S, S), lambda i, et_ref: (0, 0))),
    )
    part, attn = pl.pallas_call(
        _tc_body,
        grid_spec=grid_spec,
        out_shape=(jax.ShapeDtypeStruct((S, H), jnp.float32),
                   jax.ShapeDtypeStruct((S, S), jnp.float32)),
    )(et, global_features, Wq, Wk, *([rgcn_weight] * TC_E),
      rgcn_root, rgcn_bias.reshape(1, H))

    return pl.pallas_call(
        _combine_body,
        out_shape=jax.ShapeDtypeStruct((S, 2 * H), jnp.float32),
    )(part, attn, raw, global_features, gcn_lin_rel, gcn_lin_root,
      gcn_bias.reshape(1, H))


# speedup vs baseline: 3.4125x; 2.1868x over previous
import jax
import jax.numpy as jnp
from jax.experimental import pallas as pl
from jax.experimental.pallas import tpu as pltpu

S = 8
H = 256
E = S * S


def _body(et_ref, gf_ref, wq_ref, wk_ref, *rest):
    w_refs = rest[:2 * E]
    root_ref, rb_ref, lrel_ref, lroot_ref, gb_ref, out_ref = rest[2 * E:]

    gf = gf_ref[...]
    q = jnp.dot(gf, wq_ref[...], preferred_element_type=jnp.float32)
    k = jnp.dot(gf, wk_ref[...], preferred_element_type=jnp.float32)
    scores = jnp.dot(q, k.T, preferred_element_type=jnp.float32) * (1.0 / 16.0)
    scores = scores - jnp.max(scores, axis=-1, keepdims=True)
    ex = jnp.exp(scores)
    attn = ex / jnp.sum(ex, axis=-1, keepdims=True)
    wgf = (attn[:, :, None] * gf[:, None, :]).reshape(E, H)

    msgs = [jnp.dot(wgf[e:e + 1, :128], w_refs[2 * e][0],
                    preferred_element_type=jnp.float32)
            + jnp.dot(wgf[e:e + 1, 128:], w_refs[2 * e + 1][0],
                      preferred_element_type=jnp.float32) for e in range(E)]
    x1 = (jnp.dot(gf, root_ref[...], preferred_element_type=jnp.float32)
          + rb_ref[...])
    for b in range(S):
        x1 = x1 + jnp.concatenate(msgs[S * b:S * (b + 1)], axis=0)

    agg = jnp.broadcast_to(jnp.sum(x1, axis=0, keepdims=True), (S, H))
    out2 = (jnp.dot(agg, lrel_ref[...], preferred_element_type=jnp.float32)
            + jnp.dot(x1, lroot_ref[...], preferred_element_type=jnp.float32)
            + gb_ref[...])
    out_ref[:, :H] = out2
    out_ref[:, H:] = gf


def kernel(global_features, speaker, Wq, Wk, Wv, rgcn_weight, rgcn_root,
           rgcn_bias, gcn_lin_rel, gcn_lin_root, gcn_bias):
    del Wv
    spk = speaker.astype(jnp.int32)
    src = jnp.repeat(jnp.arange(S, dtype=jnp.int32), S)
    dst = jnp.tile(jnp.arange(S, dtype=jnp.int32), S)
    et = (spk[src] * S + spk[dst]) * 2 + (src >= dst).astype(jnp.int32)

    full = lambda shape: pl.BlockSpec(shape, lambda i, et_ref: (0,) * len(shape))
    w_specs = [pl.BlockSpec((1, H // 2, H),
                            lambda i, et_ref, e=e, r=r: (et_ref[e], r, 0))
               for e in range(E) for r in range(2)]
    grid_spec = pltpu.PrefetchScalarGridSpec(
        num_scalar_prefetch=1,
        grid=(1,),
        in_specs=[
            full((S, H)),
            full((H, H)),
            full((H, H)),
            *w_specs,
            full((H, H)),
            full((1, H)),
            full((H, H)),
            full((H, H)),
            full((1, H)),
        ],
        out_specs=pl.BlockSpec((S, 2 * H), lambda i, et_ref: (0, 0)),
    )
    return pl.pallas_call(
        _body,
        grid_spec=grid_spec,
        out_shape=jax.ShapeDtypeStruct((S, 2 * H), jnp.float32),
    )(et, global_features, Wq, Wk, *([rgcn_weight] * (2 * E)), rgcn_root,
      rgcn_bias.reshape(1, H), gcn_lin_rel, gcn_lin_root,
      gcn_bias.reshape(1, H))


# speedup vs baseline: 3.4354x; 1.0067x over previous
import jax
import jax.numpy as jnp
from jax.experimental import pallas as pl
from jax.experimental.pallas import tpu as pltpu

S = 8
H = 256
E = S * S


def _body(et_ref, gf_ref, wq_ref, wk_ref, *rest):
    w_refs = rest[:E]
    root_ref, rb_ref, lrel_ref, lroot_ref, gb_ref, out_ref = rest[E:]

    gf = gf_ref[...]
    q = jnp.dot(gf, wq_ref[...], preferred_element_type=jnp.float32)
    k = jnp.dot(gf, wk_ref[...], preferred_element_type=jnp.float32)
    scores = jnp.dot(q, k.T, preferred_element_type=jnp.float32) * (1.0 / 16.0)
    scores = scores - jnp.max(scores, axis=-1, keepdims=True)
    ex = jnp.exp(scores)
    attn = ex / jnp.sum(ex, axis=-1, keepdims=True)
    wgf = (attn[:, :, None] * gf[:, None, :]).reshape(E, H)

    msgs = [jnp.dot(wgf[e:e + 1, :], w_refs[e][0],
                    preferred_element_type=jnp.float32) for e in range(E)]
    x1 = (jnp.dot(gf, root_ref[...], preferred_element_type=jnp.float32)
          + rb_ref[...])
    for b in range(S):
        x1 = x1 + jnp.concatenate(msgs[S * b:S * (b + 1)], axis=0)

    agg = jnp.broadcast_to(jnp.sum(x1, axis=0, keepdims=True), (S, H))
    out2 = (jnp.dot(agg, lrel_ref[...], preferred_element_type=jnp.float32)
            + jnp.dot(x1, lroot_ref[...], preferred_element_type=jnp.float32)
            + gb_ref[...])
    out_ref[:, :H] = out2
    out_ref[:, H:] = gf


def kernel(global_features, speaker, Wq, Wk, Wv, rgcn_weight, rgcn_root,
           rgcn_bias, gcn_lin_rel, gcn_lin_root, gcn_bias):
    del Wv
    spk = speaker.astype(jnp.int32)
    src = jnp.repeat(jnp.arange(S, dtype=jnp.int32), S)
    dst = jnp.tile(jnp.arange(S, dtype=jnp.int32), S)
    et = (spk[src] * S + spk[dst]) * 2 + (src >= dst).astype(jnp.int32)

    full = lambda shape: pl.BlockSpec(shape, lambda i, et_ref: (0,) * len(shape))
    w_specs = [pl.BlockSpec((1, H, H),
                            lambda i, et_ref, e=e: (et_ref[e], 0, 0))
               for e in range(E)]
    grid_spec = pltpu.PrefetchScalarGridSpec(
        num_scalar_prefetch=1,
        grid=(1,),
        in_specs=[
            full((S, H)),
            full((H, H)),
            full((H, H)),
            *w_specs,
            full((H, H)),
            full((1, H)),
            full((H, H)),
            full((H, H)),
            full((1, H)),
        ],
        out_specs=pl.BlockSpec((S, 2 * H), lambda i, et_ref: (0, 0)),
    )
    return pl.pallas_call(
        _body,
        grid_spec=grid_spec,
        out_shape=jax.ShapeDtypeStruct((S, 2 * H), jnp.float32),
    )(et, global_features, Wq, Wk, *([rgcn_weight] * E), rgcn_root,
      rgcn_bias.reshape(1, H), gcn_lin_rel, gcn_lin_root,
      gcn_bias.reshape(1, H))
